# baseline (speedup 1.0000x reference)
import jax
import jax.numpy as jnp
from jax.experimental import pallas as pl

_K = 100000
_B = 16384
_D = 64
_CB = 16384
_VB = _B // _CB
_NB = pl.cdiv(_K, _CB)


def _push_body(mem_ref, val_ref, out_ref):
    i = pl.program_id(0)

    @pl.when(i < _VB)
    def _():
        out_ref[...] = val_ref[...]

    @pl.when(i >= _VB)
    def _():
        out_ref[...] = mem_ref[...]


def kernel(mem, value):
    out_t = pl.pallas_call(
        _push_body,
        grid=(_NB,),
        in_specs=[
            pl.BlockSpec((_D, _CB), lambda i: (0, jnp.maximum(i, _VB))),
            pl.BlockSpec((_D, _CB), lambda i: (0, jnp.minimum(i, _VB - 1))),
        ],
        out_specs=pl.BlockSpec((_D, _CB), lambda i: (0, i)),
        out_shape=jax.ShapeDtypeStruct((_D, _K), jnp.float32),
    )(mem.T, value.T)
    return out_t.T
